# Initial kernel scaffold; baseline (speedup 1.0000x reference)
import functools

import jax
import jax.numpy as jnp
from jax import lax
from jax.experimental import pallas as pl
from jax.experimental.pallas import tpu as pltpu
from jax.experimental.pallas import tpu_sc as plsc

N = 10000
E = 320000
D = 128

NC = 2
NS = 16
CHUNK = 80
E_PER_SUB = E // (NC * NS)
N_ITERS = E_PER_SUB // CHUNK
ROWS_PER_SUB = N // NS
ZROWS = 125

_MESH = plsc.VectorSubcoreMesh(core_axis_name="c", subcore_axis_name="s")


def _zero_fill(ref, nrows, ncols):
    def row(i, _):
        def col(j, _):
            ref[i, pl.ds(j * 16, 16)] = jnp.zeros((16,), jnp.float32)
            return 0
        lax.fori_loop(0, ncols // 16, col, 0)
        return 0
    lax.fori_loop(0, nrows, row, 0)


@functools.partial(
    pl.kernel,
    out_type=(jax.ShapeDtypeStruct((NC * N, 16), jnp.float32),
              jax.ShapeDtypeStruct((NC * N, 16), jnp.float32)),
    mesh=_MESH,
    scratch_types=[
        pltpu.VMEM_SHARED((N, 16), jnp.float32),
        pltpu.VMEM_SHARED((N, 16), jnp.float32),
        pltpu.VMEM((CHUNK,), jnp.int32),
        pltpu.VMEM((CHUNK,), jnp.int32),
        pltpu.VMEM((CHUNK, 16), jnp.float32),
        pltpu.VMEM((ZROWS, 16), jnp.float32),
    ],
)
def _sc_degrees(src_hbm, dst_hbm, din_hbm, dout_hbm,
                acc_in, acc_out, idx_s, idx_d, ones_v, zeros_v):
    c = lax.axis_index("c")
    s = lax.axis_index("s")

    def fill_ones(i, _):
        ones_v[i, :] = jnp.full((16,), 1.0, jnp.float32)
        return 0
    lax.fori_loop(0, CHUNK, fill_ones, 0)
    _zero_fill(zeros_v, ZROWS, 16)

    row0 = s * ROWS_PER_SUB
    for j in range(ROWS_PER_SUB // ZROWS):
        pltpu.sync_copy(zeros_v, acc_in.at[pl.ds(row0 + j * ZROWS, ZROWS)])
        pltpu.sync_copy(zeros_v, acc_out.at[pl.ds(row0 + j * ZROWS, ZROWS)])
    plsc.subcore_barrier()

    e0 = c * (E // NC) + s * E_PER_SUB

    def body(i, _):
        off = e0 + i * CHUNK
        pltpu.sync_copy(src_hbm.at[pl.ds(off, CHUNK)], idx_s)
        pltpu.sync_copy(dst_hbm.at[pl.ds(off, CHUNK)], idx_d)
        pltpu.sync_copy(ones_v, acc_out.at[idx_s], add=True)
        pltpu.sync_copy(ones_v, acc_in.at[idx_d], add=True)
        return 0
    lax.fori_loop(0, N_ITERS, body, 0)
    plsc.subcore_barrier()

    out0 = c * N + row0
    pltpu.sync_copy(acc_in.at[pl.ds(row0, ROWS_PER_SUB)],
                    din_hbm.at[pl.ds(out0, ROWS_PER_SUB)])
    pltpu.sync_copy(acc_out.at[pl.ds(row0, ROWS_PER_SUB)],
                    dout_hbm.at[pl.ds(out0, ROWS_PER_SUB)])


@functools.partial(
    pl.kernel,
    out_type=jax.ShapeDtypeStruct((NC * N, D), jnp.float32),
    mesh=_MESH,
    scratch_types=[
        pltpu.VMEM_SHARED((N, D), jnp.float32),
        pltpu.VMEM((CHUNK,), jnp.int32),
        pltpu.VMEM((CHUNK,), jnp.int32),
        pltpu.VMEM((CHUNK, D), jnp.float32),
        pltpu.VMEM((ZROWS, D), jnp.float32),
        pltpu.SemaphoreType.DMA,
    ],
)
def _sc_scatter(y_hbm, src_hbm, dst_hbm, out_hbm,
                acc, idx_s, idx_d, rows, zeros_v, sem):
    c = lax.axis_index("c")
    s = lax.axis_index("s")

    _zero_fill(zeros_v, ZROWS, D)
    row0 = s * ROWS_PER_SUB
    for j in range(ROWS_PER_SUB // ZROWS):
        pltpu.sync_copy(zeros_v, acc.at[pl.ds(row0 + j * ZROWS, ZROWS)])
    plsc.subcore_barrier()

    e0 = c * (E // NC) + s * E_PER_SUB

    def body(i, _):
        off = e0 + i * CHUNK
        pltpu.sync_copy(src_hbm.at[pl.ds(off, CHUNK)], idx_s)
        pltpu.sync_copy(dst_hbm.at[pl.ds(off, CHUNK)], idx_d)
        pltpu.async_copy(y_hbm.at[idx_s], rows, sem).wait()
        pltpu.sync_copy(rows, acc.at[idx_d], add=True)
        return 0
    lax.fori_loop(0, N_ITERS, body, 0)
    plsc.subcore_barrier()

    pltpu.sync_copy(acc.at[pl.ds(row0, ROWS_PER_SUB)],
                    out_hbm.at[pl.ds(c * N + row0, ROWS_PER_SUB)])


def _norm_from_deg_parts(deg_ref):
    deg = deg_ref[0:N, 0:1] + deg_ref[N:2 * N, 0:1]
    return jnp.where(deg > 0, 1.0 / jnp.sqrt(deg), 0.0)


def _tc_prescale_body(x_ref, do_ref, o_ref):
    o_ref[:, :] = x_ref[:, :] * _norm_from_deg_parts(do_ref)


def _tc_layer_body(p_ref, di_ref, do_ref, w_ref, b_ref, o_ref,
                   *, relu, scale_src):
    agg = (p_ref[0:N, :] + p_ref[N:2 * N, :]) * _norm_from_deg_parts(di_ref)
    out = jnp.dot(agg, w_ref[:, :], preferred_element_type=jnp.float32)
    out = out + b_ref[:, :]
    if relu:
        out = jnp.maximum(out, 0.0)
    if scale_src:
        out = out * _norm_from_deg_parts(do_ref)
    o_ref[:, :] = out


def _tc_prescale(x, dout_p):
    return pl.pallas_call(
        _tc_prescale_body,
        out_shape=jax.ShapeDtypeStruct((N, D), jnp.float32),
    )(x, dout_p)


def _tc_layer(part, din_p, dout_p, W, b, relu, scale_src):
    body = functools.partial(_tc_layer_body, relu=relu, scale_src=scale_src)
    return pl.pallas_call(
        body,
        out_shape=jax.ShapeDtypeStruct((N, D), jnp.float32),
    )(part, din_p, dout_p, W, b.reshape(1, D))


def kernel(x, edge_index, W1, b1, W2, b2, W3, b3):
    src = edge_index[0].astype(jnp.int32)
    dst = edge_index[1].astype(jnp.int32)

    din_p, dout_p = _sc_degrees(src, dst)
    y = _tc_prescale(x, dout_p)
    for W, b, last in ((W1, b1, False), (W2, b2, False), (W3, b3, True)):
        part = _sc_scatter(y, src, dst)
        y = _tc_layer(part, din_p, dout_p, W, b,
                      relu=not last, scale_src=not last)
    return y


# speedup vs baseline: 4.8050x; 4.8050x over previous
import functools

import jax
import jax.numpy as jnp
from jax import lax
from jax.experimental import pallas as pl
from jax.experimental.pallas import tpu as pltpu
from jax.experimental.pallas import tpu_sc as plsc

N = 10000
E = 320000
D = 128

NC = 2
NS = 16
CHUNK = 80
E_PER_SUB = E // (NC * NS)
N_ITERS = E_PER_SUB // CHUNK
ROWS_PER_SUB = 624
TAIL_ROW0 = NS * ROWS_PER_SUB
TAIL_ROWS = N - TAIL_ROW0
ZROWS = 208

_MESH = plsc.VectorSubcoreMesh(core_axis_name="c", subcore_axis_name="s",
                               num_cores=NC, num_subcores=NS)


def _zero_fill(ref, nrows, ncols):
    def row(i, _):
        def col(j, _):
            ref[i, pl.ds(j * 16, 16)] = jnp.zeros((16,), jnp.float32)
            return 0
        lax.fori_loop(0, ncols // 16, col, 0)
        return 0
    lax.fori_loop(0, nrows, row, 0)


def _sc_degrees_body(src_hbm, dst_hbm, din_hbm, dout_hbm,
                acc_in, acc_out, idx_s, idx_d, ones_v, zeros_v):
    c = lax.axis_index("c")
    s = lax.axis_index("s")

    def fill_ones(i, _):
        ones_v[i, :] = jnp.full((16,), 1.0, jnp.float32)
        return 0
    lax.fori_loop(0, CHUNK, fill_ones, 0)
    _zero_fill(zeros_v, ZROWS, 16)

    row0 = s * ROWS_PER_SUB
    for j in range(ROWS_PER_SUB // ZROWS):
        pltpu.sync_copy(zeros_v, acc_in.at[pl.ds(row0 + j * ZROWS, ZROWS)])
        pltpu.sync_copy(zeros_v, acc_out.at[pl.ds(row0 + j * ZROWS, ZROWS)])

    @pl.when(s == NS - 1)
    def _():
        pltpu.sync_copy(zeros_v.at[pl.ds(0, TAIL_ROWS)],
                        acc_in.at[pl.ds(TAIL_ROW0, TAIL_ROWS)])
        pltpu.sync_copy(zeros_v.at[pl.ds(0, TAIL_ROWS)],
                        acc_out.at[pl.ds(TAIL_ROW0, TAIL_ROWS)])
    plsc.subcore_barrier()

    e0 = c * (E // NC) + s * E_PER_SUB

    def body(i, _):
        off = e0 + i * CHUNK
        pltpu.sync_copy(src_hbm.at[pl.ds(off, CHUNK)], idx_s)
        pltpu.sync_copy(dst_hbm.at[pl.ds(off, CHUNK)], idx_d)
        pltpu.sync_copy(ones_v, acc_out.at[idx_s], add=True)
        pltpu.sync_copy(ones_v, acc_in.at[idx_d], add=True)
        return 0
    lax.fori_loop(0, N_ITERS, body, 0)
    plsc.subcore_barrier()

    out0 = c * N + row0
    pltpu.sync_copy(acc_in.at[pl.ds(row0, ROWS_PER_SUB)],
                    din_hbm.at[pl.ds(out0, ROWS_PER_SUB)])
    pltpu.sync_copy(acc_out.at[pl.ds(row0, ROWS_PER_SUB)],
                    dout_hbm.at[pl.ds(out0, ROWS_PER_SUB)])

    @pl.when(s == NS - 1)
    def _():
        pltpu.sync_copy(acc_in.at[pl.ds(TAIL_ROW0, TAIL_ROWS)],
                        din_hbm.at[pl.ds(c * N + TAIL_ROW0, TAIL_ROWS)])
        pltpu.sync_copy(acc_out.at[pl.ds(TAIL_ROW0, TAIL_ROWS)],
                        dout_hbm.at[pl.ds(c * N + TAIL_ROW0, TAIL_ROWS)])


def _sc_scatter_body(y_hbm, src_hbm, dst_hbm, out_hbm,
                acc, idx_s, idx_d, rows, zeros_v, sem):
    c = lax.axis_index("c")
    s = lax.axis_index("s")

    _zero_fill(zeros_v, ZROWS, D)
    row0 = s * ROWS_PER_SUB
    for j in range(ROWS_PER_SUB // ZROWS):
        pltpu.sync_copy(zeros_v, acc.at[pl.ds(row0 + j * ZROWS, ZROWS)])

    @pl.when(s == NS - 1)
    def _():
        pltpu.sync_copy(zeros_v.at[pl.ds(0, TAIL_ROWS)],
                        acc.at[pl.ds(TAIL_ROW0, TAIL_ROWS)])
    plsc.subcore_barrier()

    e0 = c * (E // NC) + s * E_PER_SUB

    def body(i, _):
        off = e0 + i * CHUNK
        pltpu.sync_copy(src_hbm.at[pl.ds(off, CHUNK)], idx_s)
        pltpu.sync_copy(dst_hbm.at[pl.ds(off, CHUNK)], idx_d)
        pltpu.async_copy(y_hbm.at[idx_s], rows, sem).wait()
        pltpu.sync_copy(rows, acc.at[idx_d], add=True)
        return 0
    lax.fori_loop(0, N_ITERS, body, 0)
    plsc.subcore_barrier()

    pltpu.sync_copy(acc.at[pl.ds(row0, ROWS_PER_SUB)],
                    out_hbm.at[pl.ds(c * N + row0, ROWS_PER_SUB)])

    @pl.when(s == NS - 1)
    def _():
        pltpu.sync_copy(acc.at[pl.ds(TAIL_ROW0, TAIL_ROWS)],
                        out_hbm.at[pl.ds(c * N + TAIL_ROW0, TAIL_ROWS)])


def _make_sc_kernels(interpret=False):
    deg = pl.kernel(
        _sc_degrees_body,
        out_type=(jax.ShapeDtypeStruct((NC * N, 16), jnp.float32),
                  jax.ShapeDtypeStruct((NC * N, 16), jnp.float32)),
        mesh=_MESH,
        scratch_types=[
            pltpu.VMEM_SHARED((N, 16), jnp.float32),
            pltpu.VMEM_SHARED((N, 16), jnp.float32),
            pltpu.VMEM((CHUNK,), jnp.int32),
            pltpu.VMEM((CHUNK,), jnp.int32),
            pltpu.VMEM((CHUNK, 16), jnp.float32),
            pltpu.VMEM((ZROWS, 16), jnp.float32),
        ],
        compiler_params=pltpu.CompilerParams(use_tc_tiling_on_sc=False),
        interpret=interpret,
    )
    sca = pl.kernel(
        _sc_scatter_body,
        out_type=jax.ShapeDtypeStruct((NC * N, D), jnp.float32),
        mesh=_MESH,
        scratch_types=[
            pltpu.VMEM_SHARED((N, D), jnp.float32),
            pltpu.VMEM((CHUNK,), jnp.int32),
            pltpu.VMEM((CHUNK,), jnp.int32),
            pltpu.VMEM((CHUNK, D), jnp.float32),
            pltpu.VMEM((ZROWS, D), jnp.float32),
            pltpu.SemaphoreType.DMA,
        ],
        interpret=interpret,
    )
    return deg, sca


_sc_degrees, _sc_scatter = _make_sc_kernels()


def _norm_from_deg_parts(deg_ref):
    deg = deg_ref[0:N, 0:1] + deg_ref[N:2 * N, 0:1]
    return jnp.where(deg > 0, 1.0 / jnp.sqrt(deg), 0.0)


def _tc_prescale_body(x_ref, do_ref, o_ref):
    o_ref[:, :] = x_ref[:, :] * _norm_from_deg_parts(do_ref)


def _tc_layer_body(p_ref, di_ref, do_ref, w_ref, b_ref, o_ref,
                   *, relu, scale_src):
    agg = (p_ref[0:N, :] + p_ref[N:2 * N, :]) * _norm_from_deg_parts(di_ref)
    out = jnp.dot(agg, w_ref[:, :], preferred_element_type=jnp.float32)
    out = out + b_ref[:, :]
    if relu:
        out = jnp.maximum(out, 0.0)
    if scale_src:
        out = out * _norm_from_deg_parts(do_ref)
    o_ref[:, :] = out


def _tc_prescale(x, dout_p):
    return pl.pallas_call(
        _tc_prescale_body,
        out_shape=jax.ShapeDtypeStruct((N, D), jnp.float32),
    )(x, dout_p)


def _tc_layer(part, din_p, dout_p, W, b, relu, scale_src):
    body = functools.partial(_tc_layer_body, relu=relu, scale_src=scale_src)
    return pl.pallas_call(
        body,
        out_shape=jax.ShapeDtypeStruct((N, D), jnp.float32),
    )(part, din_p, dout_p, W, b.reshape(1, D))


def kernel(x, edge_index, W1, b1, W2, b2, W3, b3):
    src = edge_index[0].astype(jnp.int32)
    dst = edge_index[1].astype(jnp.int32)

    din_p, dout_p = _sc_degrees(src, dst)
    y = _tc_prescale(x, dout_p)
    for W, b, last in ((W1, b1, False), (W2, b2, False), (W3, b3, True)):
        part = _sc_scatter(y, src, dst)
        y = _tc_layer(part, din_p, dout_p, W, b,
                      relu=not last, scale_src=not last)
    return y


# speedup vs baseline: 11.5400x; 2.4017x over previous
import functools

import jax
import jax.numpy as jnp
from jax import lax
from jax.experimental import pallas as pl
from jax.experimental.pallas import tpu as pltpu
from jax.experimental.pallas import tpu_sc as plsc

N = 10000
E = 320000
D = 128

NC = 2
NS = 16
NW = NC * NS

CH = 128
NROWS_E = E // CH
MAIN_CH = NROWS_E // NW
TAIL_CH = NROWS_E - MAIN_CH * NW

ROWS_PER_SUB = 624
TAIL_ROW0 = NS * ROWS_PER_SUB
TAIL_ROWS = N - TAIL_ROW0
ZROWS = 48

_MESH = plsc.VectorSubcoreMesh(core_axis_name="c", subcore_axis_name="s",
                               num_cores=NC, num_subcores=NS)
_SC_PARAMS = pltpu.CompilerParams(use_tc_tiling_on_sc=False)


def _zero_fill(ref, nrows, ncols):
    def row(i, _):
        def col(j, _):
            ref[i, pl.ds(j * 16, 16)] = jnp.zeros((16,), jnp.float32)
            return 0
        lax.fori_loop(0, ncols // 16, col, 0)
        return 0
    lax.fori_loop(0, nrows, row, 0)


def _sc_degrees_body(src_hbm, dst_hbm, din_hbm, dout_hbm,
                     acc_in, acc_out, src_v, dst_v, ones_v, zeros_v, sem):
    c = lax.axis_index("c")
    s = lax.axis_index("s")
    w = c * NS + s

    pltpu.sync_copy(src_hbm.at[pl.ds(w * MAIN_CH, MAIN_CH)],
                    src_v.at[pl.ds(0, MAIN_CH)])
    pltpu.sync_copy(dst_hbm.at[pl.ds(w * MAIN_CH, MAIN_CH)],
                    dst_v.at[pl.ds(0, MAIN_CH)])

    @pl.when(w < TAIL_CH)
    def _():
        pltpu.sync_copy(src_hbm.at[pl.ds(NW * MAIN_CH + w, 1)],
                        src_v.at[pl.ds(MAIN_CH, 1)])
        pltpu.sync_copy(dst_hbm.at[pl.ds(NW * MAIN_CH + w, 1)],
                        dst_v.at[pl.ds(MAIN_CH, 1)])

    def fill_ones(i, _):
        ones_v[i, :] = jnp.full((16,), 1.0, jnp.float32)
        return 0
    lax.fori_loop(0, CH, fill_ones, 0)
    _zero_fill(zeros_v, ZROWS, 16)

    row0 = s * ROWS_PER_SUB
    for j in range(ROWS_PER_SUB // ZROWS):
        pltpu.sync_copy(zeros_v, acc_in.at[pl.ds(row0 + j * ZROWS, ZROWS)])
        pltpu.sync_copy(zeros_v, acc_out.at[pl.ds(row0 + j * ZROWS, ZROWS)])

    @pl.when(s == NS - 1)
    def _():
        pltpu.sync_copy(zeros_v.at[pl.ds(0, TAIL_ROWS)],
                        acc_in.at[pl.ds(TAIL_ROW0, TAIL_ROWS)])
        pltpu.sync_copy(zeros_v.at[pl.ds(0, TAIL_ROWS)],
                        acc_out.at[pl.ds(TAIL_ROW0, TAIL_ROWS)])
    plsc.subcore_barrier()

    def group(g, _):
        for j in range(2):
            k = g * 2 + j
            pltpu.async_copy(ones_v, acc_out.at[src_v.at[k]], sem, add=True)
            pltpu.async_copy(ones_v, acc_in.at[dst_v.at[k]], sem, add=True)
        for j in range(2):
            k = g * 2 + j
            pltpu.make_async_copy(ones_v, acc_out.at[src_v.at[k]], sem).wait()
            pltpu.make_async_copy(ones_v, acc_in.at[dst_v.at[k]], sem).wait()
        return 0
    lax.fori_loop(0, MAIN_CH // 2, group, 0)

    @pl.when(w < TAIL_CH)
    def _():
        pltpu.sync_copy(ones_v, acc_out.at[src_v.at[MAIN_CH]], add=True)
        pltpu.sync_copy(ones_v, acc_in.at[dst_v.at[MAIN_CH]], add=True)
    plsc.subcore_barrier()

    out0 = c * N + row0
    pltpu.sync_copy(acc_in.at[pl.ds(row0, ROWS_PER_SUB)],
                    din_hbm.at[pl.ds(out0, ROWS_PER_SUB)])
    pltpu.sync_copy(acc_out.at[pl.ds(row0, ROWS_PER_SUB)],
                    dout_hbm.at[pl.ds(out0, ROWS_PER_SUB)])

    @pl.when(s == NS - 1)
    def _():
        pltpu.sync_copy(acc_in.at[pl.ds(TAIL_ROW0, TAIL_ROWS)],
                        din_hbm.at[pl.ds(c * N + TAIL_ROW0, TAIL_ROWS)])
        pltpu.sync_copy(acc_out.at[pl.ds(TAIL_ROW0, TAIL_ROWS)],
                        dout_hbm.at[pl.ds(c * N + TAIL_ROW0, TAIL_ROWS)])


def _sc_scatter_body(y_hbm, src_hbm, dst_hbm, out_hbm,
                     acc, is0, is1, id0, id1, rows0, rows1, zeros_v,
                     semg0, semg1, semi0, semi1, semz):
    c = lax.axis_index("c")
    s = lax.axis_index("s")
    w = c * NS + s
    row0 = s * ROWS_PER_SUB

    def idx_start(k, sv, dv, sem):
        pltpu.make_async_copy(src_hbm.at[k], sv, sem).start()
        pltpu.make_async_copy(dst_hbm.at[k], dv, sem).start()

    def idx_wait(k, sv, dv, sem):
        pltpu.make_async_copy(src_hbm.at[k], sv, sem).wait()
        pltpu.make_async_copy(dst_hbm.at[k], dv, sem).wait()

    base = w * MAIN_CH
    idx_start(base, is0, id0, semi0)
    idx_start(base + 1, is1, id1, semi1)

    _zero_fill(zeros_v, ZROWS, D)
    nz = ROWS_PER_SUB // ZROWS
    for j in range(nz):
        pltpu.async_copy(zeros_v, acc.at[pl.ds(row0 + j * ZROWS, ZROWS)],
                         semz)

    @pl.when(s == NS - 1)
    def _():
        pltpu.async_copy(zeros_v.at[pl.ds(0, TAIL_ROWS)],
                         acc.at[pl.ds(TAIL_ROW0, TAIL_ROWS)], semz)
    for j in range(nz):
        pltpu.make_async_copy(
            zeros_v, acc.at[pl.ds(row0 + j * ZROWS, ZROWS)], semz).wait()

    @pl.when(s == NS - 1)
    def _():
        pltpu.make_async_copy(
            zeros_v.at[pl.ds(0, TAIL_ROWS)],
            acc.at[pl.ds(TAIL_ROW0, TAIL_ROWS)], semz).wait()

    idx_wait(base, is0, id0, semi0)
    pltpu.make_async_copy(y_hbm.at[is0], rows0, semg0).start()
    plsc.subcore_barrier()

    def step(k2, _):
        a = 2 * k2
        b = a + 1
        idx_wait(base + b, is1, id1, semi1)
        pltpu.make_async_copy(y_hbm.at[is1], rows1, semg1).start()
        pltpu.make_async_copy(y_hbm.at[is0], rows0, semg0).wait()
        pltpu.sync_copy(rows0, acc.at[id0], add=True)

        @pl.when(a + 2 < MAIN_CH)
        def _():
            idx_start(base + a + 2, is0, id0, semi0)
            idx_wait(base + a + 2, is0, id0, semi0)
            pltpu.make_async_copy(y_hbm.at[is0], rows0, semg0).start()
        pltpu.make_async_copy(y_hbm.at[is1], rows1, semg1).wait()
        pltpu.sync_copy(rows1, acc.at[id1], add=True)

        @pl.when(b + 2 < MAIN_CH)
        def _():
            idx_start(base + b + 2, is1, id1, semi1)
        return 0
    lax.fori_loop(0, MAIN_CH // 2, step, 0)

    @pl.when(w < TAIL_CH)
    def _():
        t = NW * MAIN_CH + w
        idx_start(t, is0, id0, semi0)
        idx_wait(t, is0, id0, semi0)
        pltpu.async_copy(y_hbm.at[is0], rows0, semg0).wait()
        pltpu.sync_copy(rows0, acc.at[id0], add=True)
    plsc.subcore_barrier()

    pltpu.sync_copy(acc.at[pl.ds(row0, ROWS_PER_SUB)],
                    out_hbm.at[pl.ds(c * N + row0, ROWS_PER_SUB)])

    @pl.when(s == NS - 1)
    def _():
        pltpu.sync_copy(acc.at[pl.ds(TAIL_ROW0, TAIL_ROWS)],
                        out_hbm.at[pl.ds(c * N + TAIL_ROW0, TAIL_ROWS)])


def _make_sc_kernels(interpret=False):
    deg = pl.kernel(
        _sc_degrees_body,
        out_type=(jax.ShapeDtypeStruct((NC * N, 16), jnp.float32),
                  jax.ShapeDtypeStruct((NC * N, 16), jnp.float32)),
        mesh=_MESH,
        scratch_types=[
            pltpu.VMEM_SHARED((N, 16), jnp.float32),
            pltpu.VMEM_SHARED((N, 16), jnp.float32),
            pltpu.VMEM((MAIN_CH + 1, CH), jnp.int32),
            pltpu.VMEM((MAIN_CH + 1, CH), jnp.int32),
            pltpu.VMEM((CH, 16), jnp.float32),
            pltpu.VMEM((ZROWS, 16), jnp.float32),
            pltpu.SemaphoreType.DMA,
        ],
        compiler_params=_SC_PARAMS,
        interpret=interpret,
    )
    sca = pl.kernel(
        _sc_scatter_body,
        out_type=jax.ShapeDtypeStruct((NC * N, D), jnp.float32),
        mesh=_MESH,
        scratch_types=[
            pltpu.VMEM_SHARED((N, D), jnp.float32),
            pltpu.VMEM((CH,), jnp.int32),
            pltpu.VMEM((CH,), jnp.int32),
            pltpu.VMEM((CH,), jnp.int32),
            pltpu.VMEM((CH,), jnp.int32),
            pltpu.VMEM((CH, D), jnp.float32),
            pltpu.VMEM((CH, D), jnp.float32),
            pltpu.VMEM((ZROWS, D), jnp.float32),
            pltpu.SemaphoreType.DMA,
            pltpu.SemaphoreType.DMA,
            pltpu.SemaphoreType.DMA,
            pltpu.SemaphoreType.DMA,
            pltpu.SemaphoreType.DMA,
        ],
        compiler_params=_SC_PARAMS,
        interpret=interpret,
    )
    return deg, sca


_sc_degrees, _sc_scatter = _make_sc_kernels()


def _norm_from_deg_parts(deg_ref):
    deg = deg_ref[0:N, 0:1] + deg_ref[N:2 * N, 0:1]
    return jnp.where(deg > 0, 1.0 / jnp.sqrt(deg), 0.0)


def _tc_prescale_body(x_ref, do_ref, o_ref):
    o_ref[:, :] = x_ref[:, :] * _norm_from_deg_parts(do_ref)


def _tc_layer_body(p_ref, di_ref, do_ref, w_ref, b_ref, o_ref,
                   *, relu, scale_src):
    agg = (p_ref[0:N, :] + p_ref[N:2 * N, :]) * _norm_from_deg_parts(di_ref)
    out = jnp.dot(agg, w_ref[:, :], preferred_element_type=jnp.float32)
    out = out + b_ref[:, :]
    if relu:
        out = jnp.maximum(out, 0.0)
    if scale_src:
        out = out * _norm_from_deg_parts(do_ref)
    o_ref[:, :] = out


def _tc_prescale(x, dout_p):
    return pl.pallas_call(
        _tc_prescale_body,
        out_shape=jax.ShapeDtypeStruct((N, D), jnp.float32),
    )(x, dout_p)


def _tc_layer(part, din_p, dout_p, W, b, relu, scale_src):
    body = functools.partial(_tc_layer_body, relu=relu, scale_src=scale_src)
    return pl.pallas_call(
        body,
        out_shape=jax.ShapeDtypeStruct((N, D), jnp.float32),
    )(part, din_p, dout_p, W, b.reshape(1, D))


def kernel(x, edge_index, W1, b1, W2, b2, W3, b3):
    src = edge_index[0].astype(jnp.int32).reshape(NROWS_E, CH)
    dst = edge_index[1].astype(jnp.int32).reshape(NROWS_E, CH)

    din_p, dout_p = _sc_degrees(src, dst)
    y = _tc_prescale(x, dout_p)
    for W, b, last in ((W1, b1, False), (W2, b2, False), (W3, b3, True)):
        part = _sc_scatter(y, src, dst)
        y = _tc_layer(part, din_p, dout_p, W, b,
                      relu=not last, scale_src=not last)
    return y
